# baseline (speedup 1.0000x reference)
import functools

import jax
import jax.numpy as jnp
from jax import lax
from jax.experimental import pallas as pl
from jax.experimental.pallas import tpu as pltpu
from jax.experimental.pallas import tpu_sc as plsc

N = 10000
NP = 10240
E = 320000
K = 5

NC = 2
NS = 16
NW = NC * NS
CH = 128
DR = 4
EPW = ((E + NW * CH * DR - 1) // (NW * CH * DR)) * CH * DR
EPAD = EPW * NW
RPT = NP // NS


def _softplus(v):
    return jnp.maximum(v, 0.0) + jnp.log1p(jnp.exp(-jnp.abs(v)))


def _dyn_gather(v, idx):
    return lax.gather(
        v, idx[:, None],
        lax.GatherDimensionNumbers(offset_dims=(), collapsed_slice_dims=(0,),
                                   start_index_map=(0,)),
        (1,), mode=lax.GatherScatterMode.PROMISE_IN_BOUNDS)


@functools.cache
def _make_prop(d):
    CHd = 64 if d == 128 else CH
    T = EPW // CHd
    NB = T // DR
    assert NB % 2 == 0
    mesh = plsc.VectorSubcoreMesh(core_axis_name="c", subcore_axis_name="s")

    @functools.partial(
        pl.kernel,
        out_type=jax.ShapeDtypeStruct((2, NP, d), jnp.float32),
        mesh=mesh,
        compiler_params=pltpu.CompilerParams(use_tc_tiling_on_sc=False),
        scratch_types=[pltpu.VMEM((DR, CHd), jnp.int32) for _ in range(2)]
        + [pltpu.VMEM((DR, CHd), jnp.int32) for _ in range(2)]
        + [pltpu.VMEM((DR, CHd), jnp.float32) for _ in range(2)]
        + [pltpu.VMEM((CHd, d), jnp.float32) for _ in range(DR)] + [
            pltpu.VMEM_SHARED((NP, d), jnp.float32),
        ] + [pltpu.SemaphoreType.DMA for _ in range(2 * DR + 2)],
    )
    def prop(row_h, col_h, nrm_h, x_h, z_h, out_h,
             ri0, ri1, ci0, ci1, nr0, nr1, r0, r1, r2, r3, acc,
             g0, g1, g2, g3, s0, s1, s2, s3, p0, p1):
        ridx = (ri0, ri1)
        cidx = (ci0, ci1)
        nrmb = (nr0, nr1)
        rows = (r0, r1, r2, r3)
        gsem = (g0, g1, g2, g3)
        ssem = (s0, s1, s2, s3)
        psem = (p0, p1)
        cid = lax.axis_index("c")
        sid = lax.axis_index("s")
        wid = sid * NC + cid

        def stage(k, p):
            pltpu.async_copy(row_h.at[wid, pl.ds(k * DR, DR)], ridx[p],
                             psem[p])
            pltpu.async_copy(col_h.at[wid, pl.ds(k * DR, DR)], cidx[p],
                             psem[p])
            pltpu.async_copy(nrm_h.at[wid, pl.ds(k * DR, DR)], nrmb[p],
                             psem[p])

        def stage_wait(k, p):
            pltpu.make_async_copy(row_h.at[wid, pl.ds(k * DR, DR)], ridx[p],
                                  psem[p]).wait()
            pltpu.make_async_copy(col_h.at[wid, pl.ds(k * DR, DR)], cidx[p],
                                  psem[p]).wait()
            pltpu.make_async_copy(nrm_h.at[wid, pl.ds(k * DR, DR)], nrmb[p],
                                  psem[p]).wait()

        def gath(p, b, rb):
            pltpu.async_copy(x_h.at[ridx[p].at[b]], rows[rb], gsem[rb])

        def gath_wait(p, b, rb):
            pltpu.make_async_copy(x_h.at[ridx[p].at[b]], rows[rb],
                                  gsem[rb]).wait()

        def scat_wait(rb, p):
            pltpu.make_async_copy(rows[rb], acc.at[cidx[p].at[0]],
                                  ssem[rb]).wait()

        stage(0, 0)
        pltpu.sync_copy(z_h.at[pl.ds(sid * RPT, RPT)],
                        acc.at[pl.ds(sid * RPT, RPT)])
        stage_wait(0, 0)
        plsc.subcore_barrier()
        for b in range(DR - 1):
            gath(0, b, b)

        def block(kk, p):
            k = 2 * kk + p
            for b in range(DR):
                t = k * DR + b
                bp = (b + DR - 1) % DR
                gath_wait(p, b, b)
                for g in range(CHd // 16):
                    nv16 = nrmb[p][b, pl.ds(g * 16, 16)]
                    for r in range(16):
                        i = g * 16 + r
                        sp = _dyn_gather(nv16, jnp.full((16,), r, jnp.int32))
                        for j in range(d // 16):
                            rows[b][i, pl.ds(j * 16, 16)] = (
                                rows[b][i, pl.ds(j * 16, 16)] * sp)
                pltpu.async_copy(rows[b], acc.at[cidx[p].at[b]], ssem[b],
                                 add=True)
                if b == 0:
                    @pl.when(k > 0)
                    def _():
                        scat_wait(bp, p)

                    @pl.when(k + 1 < NB)
                    def _():
                        stage(k + 1, 1 - p)
                    gath(p, DR - 1, bp)
                else:
                    scat_wait(bp, p)
                    if b == 1:
                        @pl.when(k + 1 < NB)
                        def _():
                            stage_wait(k + 1, 1 - p)

                    @pl.when(k + 1 < NB)
                    def _():
                        gath(1 - p, b - 1, bp)

        @pl.loop(0, NB // 2)
        def dblk(kk):
            block(kk, 0)
            block(kk, 1)

        scat_wait(DR - 1, 1)
        plsc.subcore_barrier()
        pltpu.sync_copy(acc.at[pl.ds(sid * RPT, RPT)],
                        out_h.at[cid, pl.ds(sid * RPT, RPT)])

    return prop


@functools.cache
def _make_prop_fs(d):
    dh = d // 2
    CHd = 64 if d == 128 else CH
    T = (EPAD // NS) // CHd
    NB = T // DR
    assert NB % 2 == 0
    mesh = plsc.VectorSubcoreMesh(core_axis_name="c", subcore_axis_name="s")

    @functools.partial(
        pl.kernel,
        out_type=jax.ShapeDtypeStruct((NP, d), jnp.float32),
        mesh=mesh,
        compiler_params=pltpu.CompilerParams(use_tc_tiling_on_sc=False),
        scratch_types=[pltpu.VMEM((DR, CHd), jnp.int32) for _ in range(2)]
        + [pltpu.VMEM((DR, CHd), jnp.int32) for _ in range(2)]
        + [pltpu.VMEM((DR, CHd), jnp.float32) for _ in range(2)]
        + [pltpu.VMEM((CHd, dh), jnp.float32) for _ in range(DR)] + [
            pltpu.VMEM_SHARED((NP, dh), jnp.float32),
            pltpu.VMEM_SHARED((NP, dh), jnp.float32),
        ] + [pltpu.SemaphoreType.DMA for _ in range(2 * DR + 2)],
    )
    def prop(row_h, col_h, nrm_h, x_h, xsp_h, z_h, out_h,
             ri0, ri1, ci0, ci1, nr0, nr1, r0, r1, r2, r3, xs, acc,
             g0, g1, g2, g3, s0, s1, s2, s3, p0, p1):
        ridx = (ri0, ri1)
        cidx = (ci0, ci1)
        nrmb = (nr0, nr1)
        rows = (r0, r1, r2, r3)
        gsem = (g0, g1, g2, g3)
        ssem = (s0, s1, s2, s3)
        psem = (p0, p1)
        cid = lax.axis_index("c")
        sid = lax.axis_index("s")

        def stage(k, p):
            pltpu.async_copy(row_h.at[sid, pl.ds(k * DR, DR)], ridx[p],
                             psem[p])
            pltpu.async_copy(col_h.at[sid, pl.ds(k * DR, DR)], cidx[p],
                             psem[p])
            pltpu.async_copy(nrm_h.at[sid, pl.ds(k * DR, DR)], nrmb[p],
                             psem[p])

        def stage_wait(k, p):
            pltpu.make_async_copy(row_h.at[sid, pl.ds(k * DR, DR)], ridx[p],
                                  psem[p]).wait()
            pltpu.make_async_copy(col_h.at[sid, pl.ds(k * DR, DR)], cidx[p],
                                  psem[p]).wait()
            pltpu.make_async_copy(nrm_h.at[sid, pl.ds(k * DR, DR)], nrmb[p],
                                  psem[p]).wait()

        def gath(p, b, rb):
            if b % 2 == 0:
                pltpu.async_copy(xs.at[ridx[p].at[b]], rows[rb], gsem[rb])
            else:
                pltpu.async_copy(xsp_h.at[cid].at[ridx[p].at[b]], rows[rb],
                                 gsem[rb])

        def gath_wait(p, b, rb):
            if b % 2 == 0:
                pltpu.make_async_copy(xs.at[ridx[p].at[b]], rows[rb],
                                      gsem[rb]).wait()
            else:
                pltpu.make_async_copy(xsp_h.at[cid].at[ridx[p].at[b]],
                                      rows[rb], gsem[rb]).wait()

        def scat_wait(rb, p):
            pltpu.make_async_copy(rows[rb], acc.at[cidx[p].at[0]],
                                  ssem[rb]).wait()

        stage(0, 0)
        rsl = pl.ds(sid * RPT, RPT)
        csl = pl.ds(cid * dh, dh)
        pltpu.sync_copy(x_h.at[rsl, csl], xs.at[rsl])
        pltpu.sync_copy(z_h.at[rsl], acc.at[rsl])
        stage_wait(0, 0)
        plsc.subcore_barrier()
        for b in range(DR - 1):
            gath(0, b, b)

        def block(kk, p):
            k = 2 * kk + p
            for b in range(DR):
                bp = (b + DR - 1) % DR
                gath_wait(p, b, b)
                for g in range(CHd // 16):
                    nv16 = nrmb[p][b, pl.ds(g * 16, 16)]
                    for r in range(16):
                        i = g * 16 + r
                        sp = _dyn_gather(nv16, jnp.full((16,), r, jnp.int32))
                        for j in range(dh // 16):
                            rows[b][i, pl.ds(j * 16, 16)] = (
                                rows[b][i, pl.ds(j * 16, 16)] * sp)
                pltpu.async_copy(rows[b], acc.at[cidx[p].at[b]], ssem[b],
                                 add=True)
                if b == 0:
                    @pl.when(k > 0)
                    def _():
                        scat_wait(bp, p)

                    @pl.when(k + 1 < NB)
                    def _():
                        stage(k + 1, 1 - p)
                    gath(p, DR - 1, bp)
                else:
                    scat_wait(bp, p)
                    if b == 1:
                        @pl.when(k + 1 < NB)
                        def _():
                            stage_wait(k + 1, 1 - p)

                    @pl.when(k + 1 < NB)
                    def _():
                        gath(1 - p, b - 1, bp)

        @pl.loop(0, NB // 2)
        def dblk(kk):
            block(kk, 0)
            block(kk, 1)

        scat_wait(DR - 1, 1)
        plsc.subcore_barrier()
        pltpu.sync_copy(acc.at[rsl], out_h.at[rsl, csl])

    return prop


@functools.cache
def _make_layer(d):
    dh = d // 2
    CHd = 64 if d == 128 else CH
    T = (EPAD // NS) // CHd
    NB = T // DR
    assert NB % 2 == 0
    NR = RPT // CHd
    assert RPT % CHd == 0
    mesh = plsc.VectorSubcoreMesh(core_axis_name="c", subcore_axis_name="s")

    @functools.partial(
        pl.kernel,
        out_type=jax.ShapeDtypeStruct((4, NP, d), jnp.float32),
        mesh=mesh,
        compiler_params=pltpu.CompilerParams(use_tc_tiling_on_sc=False),
        scratch_types=[pltpu.VMEM((DR, CHd), jnp.int32) for _ in range(2)]
        + [pltpu.VMEM((DR, CHd), jnp.int32) for _ in range(2)]
        + [pltpu.VMEM((DR, CHd), jnp.float32) for _ in range(2)]
        + [pltpu.VMEM((CHd, dh), jnp.float32) for _ in range(DR)] + [
            pltpu.VMEM_SHARED((NP, dh), jnp.float32),
            pltpu.VMEM_SHARED((NP, dh), jnp.float32),
        ] + [pltpu.SemaphoreType.DMA for _ in range(2 * DR + 3)],
    )
    def layer(row_h, col_h, nrm_h, x_h, z_h, out_h,
              ri0, ri1, ci0, ci1, nr0, nr1, r0, r1, r2, r3, sa, sb,
              g0, g1, g2, g3, s0, s1, s2, s3, p0, p1, wsem):
        ridx = (ri0, ri1)
        cidx = (ci0, ci1)
        nrmb = (nr0, nr1)
        rows = (r0, r1, r2, r3)
        gsem = (g0, g1, g2, g3)
        ssem = (s0, s1, s2, s3)
        psem = (p0, p1)
        cid = lax.axis_index("c")
        sid = lax.axis_index("s")
        rsl = pl.ds(sid * RPT, RPT)
        csl = pl.ds(cid * dh, dh)

        def stage(k, p):
            pltpu.async_copy(row_h.at[sid, pl.ds(k * DR, DR)], ridx[p],
                             psem[p])
            pltpu.async_copy(col_h.at[sid, pl.ds(k * DR, DR)], cidx[p],
                             psem[p])
            pltpu.async_copy(nrm_h.at[sid, pl.ds(k * DR, DR)], nrmb[p],
                             psem[p])

        def stage_wait(k, p):
            pltpu.make_async_copy(row_h.at[sid, pl.ds(k * DR, DR)], ridx[p],
                                  psem[p]).wait()
            pltpu.make_async_copy(col_h.at[sid, pl.ds(k * DR, DR)], cidx[p],
                                  psem[p]).wait()
            pltpu.make_async_copy(nrm_h.at[sid, pl.ds(k * DR, DR)], nrmb[p],
                                  psem[p]).wait()

        def run_prop(src, acc, factor):
            def gath(p, b, rb):
                pltpu.async_copy(src.at[ridx[p].at[b]], rows[rb], gsem[rb])

            def gath_wait(p, b, rb):
                pltpu.make_async_copy(src.at[ridx[p].at[b]], rows[rb],
                                      gsem[rb]).wait()

            def scat_wait(rb, p):
                pltpu.make_async_copy(rows[rb], acc.at[cidx[p].at[0]],
                                      ssem[rb]).wait()

            stage(0, 0)
            stage_wait(0, 0)
            for b in range(DR - 1):
                gath(0, b, b)

            def block(kk, p):
                k = 2 * kk + p
                for b in range(DR):
                    bp = (b + DR - 1) % DR
                    gath_wait(p, b, b)
                    rb = rows[b]
                    nb_ = nrmb[p]

                    @pl.loop(0, CHd // 16)
                    def sgrp(g, _b=b, _rb=rb, _nb=nb_):
                        nv16 = _nb[_b, pl.ds(g * 16, 16)]
                        if factor != 1.0:
                            nv16 = nv16 * factor
                        for r in range(16):
                            i = g * 16 + r
                            sp = _dyn_gather(nv16,
                                             jnp.full((16,), r, jnp.int32))
                            for j in range(dh // 16):
                                _rb[i, pl.ds(j * 16, 16)] = (
                                    _rb[i, pl.ds(j * 16, 16)] * sp)
                    pltpu.async_copy(rows[b], acc.at[cidx[p].at[b]], ssem[b],
                                     add=True)
                    if b == 0:
                        @pl.when(k > 0)
                        def _():
                            scat_wait(bp, p)

                        @pl.when(k + 1 < NB)
                        def _():
                            stage(k + 1, 1 - p)
                        gath(p, DR - 1, bp)
                    else:
                        scat_wait(bp, p)
                        if b == 1:
                            @pl.when(k + 1 < NB)
                            def _():
                                stage_wait(k + 1, 1 - p)

                        @pl.when(k + 1 < NB)
                        def _():
                            gath(1 - p, b - 1, bp)

            @pl.loop(0, NB // 2)
            def dblk(kk):
                block(kk, 0)
                block(kk, 1)

            scat_wait(DR - 1, 1)

        pltpu.sync_copy(x_h.at[rsl, csl], sa.at[rsl])
        pltpu.sync_copy(z_h.at[rsl], sb.at[rsl])
        plsc.subcore_barrier()

        run_prop(sa, sb, 1.0)
        plsc.subcore_barrier()

        bufs = (sa, sb)
        for step in range(3):
            cur = bufs[(step + 1) % 2]
            prev = bufs[step % 2]
            wout = pltpu.async_copy(cur.at[rsl], out_h.at[step, rsl, csl],
                                    wsem)
            @pl.loop(0, NR)
            def negq(q):
                qsl = pl.ds(sid * RPT + q * CHd, CHd)
                pltpu.sync_copy(prev.at[qsl], rows[0])
                for i in range(CHd):
                    for j in range(dh // 16):
                        rows[0][i, pl.ds(j * 16, 16)] = (
                            -rows[0][i, pl.ds(j * 16, 16)])
                pltpu.sync_copy(rows[0], prev.at[qsl])
            plsc.subcore_barrier()
            run_prop(cur, prev, 2.0)
            wout.wait()
            plsc.subcore_barrier()

        pltpu.sync_copy(bufs[0].at[rsl], out_h.at[3, rsl, csl])

    return layer


@functools.cache
def _make_norm():
    T = EPW // CH
    mesh = plsc.VectorSubcoreMesh(core_axis_name="c", subcore_axis_name="s")

    NB = T // DR

    @functools.partial(
        pl.kernel,
        out_type=jax.ShapeDtypeStruct((NW, T, CH), jnp.float32),
        mesh=mesh,
        compiler_params=pltpu.CompilerParams(use_tc_tiling_on_sc=False),
        scratch_types=[
            pltpu.VMEM((T, CH), jnp.int32),
            pltpu.VMEM((T, CH), jnp.float32),
            pltpu.VMEM((T, CH), jnp.float32),
        ] + [pltpu.VMEM((CH,), jnp.float32) for _ in range(DR)]
        + [pltpu.SemaphoreType.DMA for _ in range(DR + 1)],
    )
    def normk(row_h, ew_h, dinv_h, out_h, ridx, ewv, nout,
              v0, v1, v2, v3, g0, g1, g2, g3, isem):
        dv = (v0, v1, v2, v3)
        gsem = (g0, g1, g2, g3)
        cid = lax.axis_index("c")
        sid = lax.axis_index("s")
        wid = sid * NC + cid
        d1 = pltpu.async_copy(row_h.at[wid], ridx, isem)
        d2 = pltpu.async_copy(ew_h.at[wid], ewv, isem)
        d1.wait()
        d2.wait()

        def gath(t, b):
            pltpu.async_copy(dinv_h.at[ridx.at[t]], dv[b], gsem[b])

        for b in range(DR - 1):
            gath(b, b)

        @pl.loop(0, NB)
        def blk(k):
            for b in range(DR):
                t = k * DR + b
                bp = (b + DR - 1) % DR
                pltpu.make_async_copy(dinv_h.at[ridx.at[t]], dv[b],
                                      gsem[b]).wait()
                for g in range(CH // 16):
                    sl = pl.ds(g * 16, 16)
                    nout[t, sl] = -(dv[b][sl] * ewv[t, sl])

                @pl.when(t + DR - 1 < T)
                def _():
                    gath(t + DR - 1, bp)

        pltpu.sync_copy(nout, out_h.at[wid])

    return normk


def _mm5(axs, w, b, use_sp=True, bm=512):
    di = axs[0].shape[1]
    do = w.shape[2]

    def body(a0, a1, a2, a3, a4, wr, br, o):
        aref = (a0, a1, a2, a3, a4)
        acc = jnp.dot(aref[0][...], wr[0], preferred_element_type=jnp.float32)
        for k in range(1, K):
            acc = acc + jnp.dot(aref[k][...], wr[k],
                                preferred_element_type=jnp.float32)
        acc = acc + br[...]
        if use_sp:
            acc = _softplus(acc)
        o[...] = acc

    return pl.pallas_call(
        body,
        grid=(NP // bm,),
        in_specs=[pl.BlockSpec((bm, di), lambda i: (i, 0)) for _ in range(K)]
        + [pl.BlockSpec((K, di, do), lambda i: (0, 0, 0)),
           pl.BlockSpec((1, do), lambda i: (0, 0))],
        out_specs=pl.BlockSpec((bm, do), lambda i: (i, 0)),
        out_shape=jax.ShapeDtypeStruct((NP, do), jnp.float32),
    )(*axs, w, b.reshape(1, do))


def _mm1(a, w, b=None, use_sp=False, bm=512):
    di = a.shape[1]
    do = w.shape[1]

    def body(ar, wr, br, o):
        acc = jnp.dot(ar[...], wr[...], preferred_element_type=jnp.float32)
        acc = acc + br[...]
        if use_sp:
            acc = _softplus(acc)
        o[...] = acc

    if b is None:
        b = jnp.zeros((do,), jnp.float32)
    return pl.pallas_call(
        body,
        grid=(NP // bm,),
        in_specs=[pl.BlockSpec((bm, di), lambda i: (i, 0)),
                  pl.BlockSpec((di, do), lambda i: (0, 0)),
                  pl.BlockSpec((1, do), lambda i: (0, 0))],
        out_specs=pl.BlockSpec((bm, do), lambda i: (i, 0)),
        out_shape=jax.ShapeDtypeStruct((NP, do), jnp.float32),
    )(a, w, b.reshape(1, do))


def _comb(coefs, xs, b=None, use_sp=False, bm=1024):
    nx = len(coefs)
    d = xs[0].shape[1]
    use_bias = b is not None

    def body(*refs):
        o = refs[-1]
        acc = coefs[0] * refs[0][...]
        for j in range(1, nx):
            acc = acc + coefs[j] * refs[j][...]
        if use_bias:
            acc = acc + refs[nx][...]
        if use_sp:
            acc = _softplus(acc)
        o[...] = acc

    in_specs = [pl.BlockSpec((bm, d), lambda i: (i, 0)) for _ in range(nx)]
    args = list(xs)
    if use_bias:
        in_specs.append(pl.BlockSpec((1, d), lambda i: (0, 0)))
        args.append(b.reshape(1, d))
    return pl.pallas_call(
        body,
        grid=(NP // bm,),
        in_specs=in_specs,
        out_specs=pl.BlockSpec((bm, d), lambda i: (i, 0)),
        out_shape=jax.ShapeDtypeStruct((NP, d), jnp.float32),
    )(*args)


def _e3(a, chd=CH):
    return a.reshape(NW, EPW // chd // 2, chd) if chd == CH else a


def _es(a, chd):
    return a.reshape(NS, (EPAD // NS) // chd, chd)


def _propagate(rowp, colp, nrm, h):
    d = h.shape[1]
    if d == 16:
        args = [a.reshape(NW, EPW // CH, CH) for a in (rowp, colp, nrm)]
        part = _make_prop(16)(*args, h, jnp.zeros((NP, 16), jnp.float32))
        return _comb((1.0, 1.0), (part[0], part[1]))
    outs = []
    dep = None
    for lo in range(0, d, 128):
        w = min(128, d - lo)
        hin = h[:, lo:lo + w]
        if dep is not None:
            hin, _ = lax.optimization_barrier((hin, dep))
        chd = 64 if w == 128 else CH
        args = [_es(a, chd) for a in (rowp, colp, nrm)]
        hsp = hin.reshape(NP, 2, w // 2).transpose(1, 0, 2)
        s = _make_prop_fs(w)(*args, hin, hsp,
                             jnp.zeros((NP, w // 2), jnp.float32))
        dep = s
        outs.append(s)
    return outs[0] if len(outs) == 1 else jnp.concatenate(outs, axis=1)


def kernel(x, edge_weigth, params, edge_index, batch):
    row = edge_index[0]
    col = edge_index[1]
    pad = EPAD - E
    padidx = (jnp.arange(pad, dtype=jnp.int32) * 37) % N
    rowp = jnp.concatenate([row, padidx])
    colp = jnp.concatenate([col, padidx])
    ewp = jnp.concatenate([edge_weigth, jnp.zeros((pad,), jnp.float32)])
    r3 = rowp.reshape(NW, EPW // CH, CH)
    c3 = colp.reshape(NW, EPW // CH, CH)
    w3 = ewp.reshape(NW, EPW // CH, CH)

    x = jnp.concatenate([x, jnp.zeros((NP - N, x.shape[1]), jnp.float32)])
    ones16 = jnp.ones((NP, 16), jnp.float32)
    z16 = jnp.zeros((NP, 16), jnp.float32)
    dparts = _make_prop(16)(r3, r3, w3, ones16, z16)
    deg = dparts[0, :, 0] + dparts[1, :, 0]
    dinv = jnp.where(deg > 0, 1.0 / deg, 0.0)
    nrm = _make_norm()(r3, w3, dinv)
    nrmf = nrm.reshape(-1)
    n3 = nrm

    w0 = params["W0"]
    u = _mm1(x, w0.transpose(1, 0, 2).reshape(128, K * 16))
    u = [u[:, 16 * k:16 * (k + 1)] for k in range(K)]
    b4 = u[4]
    p = _make_prop(16)(r3, c3, n3, b4, z16)
    b3 = _comb((2.0, 2.0, 1.0), (p[0], p[1], u[3]))
    p = _make_prop(16)(r3, c3, n3, b3, z16)
    b2 = _comb((2.0, 2.0, -1.0, 1.0), (p[0], p[1], b4, u[2]))
    p = _make_prop(16)(r3, c3, n3, b2, z16)
    b1 = _comb((2.0, 2.0, -1.0, 1.0), (p[0], p[1], b3, u[1]))
    p = _make_prop(16)(r3, c3, n3, b1, z16)
    h = _comb((1.0, 1.0, -1.0, 1.0), (p[0], p[1], b2, u[0]),
              b=params["b0"], use_sp=True)

    for i in range(1, 7):
        wi = params["W%d" % i]
        tx0 = h
        di = tx0.shape[1]
        if di == 16 or di >= 128:
            tx1 = _propagate(rowp, colp, nrmf, tx0)
            s = _propagate(rowp, colp, nrmf, tx1)
            tx2 = _comb((2.0, -1.0), (s, tx0))
            s = _propagate(rowp, colp, nrmf, tx2)
            tx3 = _comb((2.0, -1.0), (s, tx1))
            s = _propagate(rowp, colp, nrmf, tx3)
            tx4 = _comb((2.0, -1.0), (s, tx2))
            txs = (tx1, tx2, tx3, tx4)
        else:
            groups = []
            dep = None
            for lo in range(0, di, 128):
                w = min(128, di - lo)
                hin = tx0[:, lo:lo + w]
                if dep is not None:
                    hin, _ = lax.optimization_barrier((hin, dep))
                chd = 64 if w == 128 else CH
                args = [_es(a, chd) for a in (rowp, colp, nrmf)]
                g = _make_layer(w)(*args, hin,
                                   jnp.zeros((NP, w // 2), jnp.float32))
                dep = g
                groups.append(g)
            if len(groups) == 1:
                txs = tuple(groups[0][k] for k in range(4))
            else:
                txs = tuple(
                    jnp.concatenate([g[k] for g in groups], axis=1)
                    for k in range(4))
        h = _mm5((tx0,) + txs, wi, params["b%d" % i], use_sp=True)

    fcw = jnp.zeros((512, 128), jnp.float32).at[:, :3].set(params["fc_w"].T)
    fcb = jnp.zeros((128,), jnp.float32).at[:3].set(params["fc_b"])
    out = _mm1(h, fcw, fcb)
    return out[:N, :3]


# speedup vs baseline: 1.0738x; 1.0738x over previous
import functools

import jax
import jax.numpy as jnp
from jax import lax
from jax.experimental import pallas as pl
from jax.experimental.pallas import tpu as pltpu
from jax.experimental.pallas import tpu_sc as plsc

N = 10000
NP = 10240
E = 320000
K = 5

NC = 2
NS = 16
NW = NC * NS
CH = 128
DR = 4
EPW = ((E + NW * CH * DR - 1) // (NW * CH * DR)) * CH * DR
EPAD = EPW * NW
RPT = NP // NS


def _softplus(v):
    return jnp.maximum(v, 0.0) + jnp.log1p(jnp.exp(-jnp.abs(v)))


def _dyn_gather(v, idx):
    return lax.gather(
        v, idx[:, None],
        lax.GatherDimensionNumbers(offset_dims=(), collapsed_slice_dims=(0,),
                                   start_index_map=(0,)),
        (1,), mode=lax.GatherScatterMode.PROMISE_IN_BOUNDS)


@functools.cache
def _make_prop(d):
    CHd = 64 if d == 128 else CH
    T = EPW // CHd
    NB = T // DR
    assert NB % 2 == 0
    mesh = plsc.VectorSubcoreMesh(core_axis_name="c", subcore_axis_name="s")

    @functools.partial(
        pl.kernel,
        out_type=jax.ShapeDtypeStruct((2, NP, d), jnp.float32),
        mesh=mesh,
        compiler_params=pltpu.CompilerParams(use_tc_tiling_on_sc=False),
        scratch_types=[pltpu.VMEM((DR, CHd), jnp.int32) for _ in range(2)]
        + [pltpu.VMEM((DR, CHd), jnp.int32) for _ in range(2)]
        + [pltpu.VMEM((DR, CHd), jnp.float32) for _ in range(2)]
        + [pltpu.VMEM((CHd, d), jnp.float32) for _ in range(DR)] + [
            pltpu.VMEM_SHARED((NP, d), jnp.float32),
        ] + [pltpu.SemaphoreType.DMA for _ in range(2 * DR + 2)],
    )
    def prop(row_h, col_h, nrm_h, x_h, z_h, out_h,
             ri0, ri1, ci0, ci1, nr0, nr1, r0, r1, r2, r3, acc,
             g0, g1, g2, g3, s0, s1, s2, s3, p0, p1):
        ridx = (ri0, ri1)
        cidx = (ci0, ci1)
        nrmb = (nr0, nr1)
        rows = (r0, r1, r2, r3)
        gsem = (g0, g1, g2, g3)
        ssem = (s0, s1, s2, s3)
        psem = (p0, p1)
        cid = lax.axis_index("c")
        sid = lax.axis_index("s")
        wid = sid * NC + cid

        def stage(k, p):
            pltpu.async_copy(row_h.at[wid, pl.ds(k * DR, DR)], ridx[p],
                             psem[p])
            pltpu.async_copy(col_h.at[wid, pl.ds(k * DR, DR)], cidx[p],
                             psem[p])
            pltpu.async_copy(nrm_h.at[wid, pl.ds(k * DR, DR)], nrmb[p],
                             psem[p])

        def stage_wait(k, p):
            pltpu.make_async_copy(row_h.at[wid, pl.ds(k * DR, DR)], ridx[p],
                                  psem[p]).wait()
            pltpu.make_async_copy(col_h.at[wid, pl.ds(k * DR, DR)], cidx[p],
                                  psem[p]).wait()
            pltpu.make_async_copy(nrm_h.at[wid, pl.ds(k * DR, DR)], nrmb[p],
                                  psem[p]).wait()

        def gath(p, b, rb):
            pltpu.async_copy(x_h.at[ridx[p].at[b]], rows[rb], gsem[rb])

        def gath_wait(p, b, rb):
            pltpu.make_async_copy(x_h.at[ridx[p].at[b]], rows[rb],
                                  gsem[rb]).wait()

        def scat_wait(rb, p):
            pltpu.make_async_copy(rows[rb], acc.at[cidx[p].at[0]],
                                  ssem[rb]).wait()

        stage(0, 0)
        pltpu.sync_copy(z_h.at[pl.ds(sid * RPT, RPT)],
                        acc.at[pl.ds(sid * RPT, RPT)])
        stage_wait(0, 0)
        plsc.subcore_barrier()
        for b in range(DR - 1):
            gath(0, b, b)

        def block(kk, p):
            k = 2 * kk + p
            for b in range(DR):
                t = k * DR + b
                bp = (b + DR - 1) % DR
                gath_wait(p, b, b)
                for g in range(CHd // 16):
                    nv16 = nrmb[p][b, pl.ds(g * 16, 16)]
                    for r in range(16):
                        i = g * 16 + r
                        sp = _dyn_gather(nv16, jnp.full((16,), r, jnp.int32))
                        for j in range(d // 16):
                            rows[b][i, pl.ds(j * 16, 16)] = (
                                rows[b][i, pl.ds(j * 16, 16)] * sp)
                pltpu.async_copy(rows[b], acc.at[cidx[p].at[b]], ssem[b],
                                 add=True)
                if b == 0:
                    @pl.when(k > 0)
                    def _():
                        scat_wait(bp, p)

                    @pl.when(k + 1 < NB)
                    def _():
                        stage(k + 1, 1 - p)
                    gath(p, DR - 1, bp)
                else:
                    scat_wait(bp, p)
                    if b == 1:
                        @pl.when(k + 1 < NB)
                        def _():
                            stage_wait(k + 1, 1 - p)

                    @pl.when(k + 1 < NB)
                    def _():
                        gath(1 - p, b - 1, bp)

        @pl.loop(0, NB // 2)
        def dblk(kk):
            block(kk, 0)
            block(kk, 1)

        scat_wait(DR - 1, 1)
        plsc.subcore_barrier()
        pltpu.sync_copy(acc.at[pl.ds(sid * RPT, RPT)],
                        out_h.at[cid, pl.ds(sid * RPT, RPT)])

    return prop


@functools.cache
def _make_prop_fs(d):
    dh = d // 2
    CHd = 64 if d == 128 else CH
    T = (EPAD // NS) // CHd
    NB = T // DR
    assert NB % 2 == 0
    mesh = plsc.VectorSubcoreMesh(core_axis_name="c", subcore_axis_name="s")

    @functools.partial(
        pl.kernel,
        out_type=jax.ShapeDtypeStruct((NP, d), jnp.float32),
        mesh=mesh,
        compiler_params=pltpu.CompilerParams(use_tc_tiling_on_sc=False),
        scratch_types=[pltpu.VMEM((DR, CHd), jnp.int32) for _ in range(2)]
        + [pltpu.VMEM((DR, CHd), jnp.int32) for _ in range(2)]
        + [pltpu.VMEM((DR, CHd), jnp.float32) for _ in range(2)]
        + [pltpu.VMEM((CHd, dh), jnp.float32) for _ in range(DR)] + [
            pltpu.VMEM_SHARED((NP, dh), jnp.float32),
            pltpu.VMEM_SHARED((NP, dh), jnp.float32),
        ] + [pltpu.SemaphoreType.DMA for _ in range(2 * DR + 2)],
    )
    def prop(row_h, col_h, nrm_h, x_h, z_h, out_h,
             ri0, ri1, ci0, ci1, nr0, nr1, r0, r1, r2, r3, xs, acc,
             g0, g1, g2, g3, s0, s1, s2, s3, p0, p1):
        ridx = (ri0, ri1)
        cidx = (ci0, ci1)
        nrmb = (nr0, nr1)
        rows = (r0, r1, r2, r3)
        gsem = (g0, g1, g2, g3)
        ssem = (s0, s1, s2, s3)
        psem = (p0, p1)
        cid = lax.axis_index("c")
        sid = lax.axis_index("s")

        def stage(k, p):
            pltpu.async_copy(row_h.at[sid, pl.ds(k * DR, DR)], ridx[p],
                             psem[p])
            pltpu.async_copy(col_h.at[sid, pl.ds(k * DR, DR)], cidx[p],
                             psem[p])
            pltpu.async_copy(nrm_h.at[sid, pl.ds(k * DR, DR)], nrmb[p],
                             psem[p])

        def stage_wait(k, p):
            pltpu.make_async_copy(row_h.at[sid, pl.ds(k * DR, DR)], ridx[p],
                                  psem[p]).wait()
            pltpu.make_async_copy(col_h.at[sid, pl.ds(k * DR, DR)], cidx[p],
                                  psem[p]).wait()
            pltpu.make_async_copy(nrm_h.at[sid, pl.ds(k * DR, DR)], nrmb[p],
                                  psem[p]).wait()

        def gath(p, b, rb):
            pltpu.async_copy(xs.at[ridx[p].at[b]], rows[rb], gsem[rb])

        def gath_wait(p, b, rb):
            pltpu.make_async_copy(xs.at[ridx[p].at[b]], rows[rb],
                                  gsem[rb]).wait()

        def scat_wait(rb, p):
            pltpu.make_async_copy(rows[rb], acc.at[cidx[p].at[0]],
                                  ssem[rb]).wait()

        stage(0, 0)
        rsl = pl.ds(sid * RPT, RPT)
        csl = pl.ds(cid * dh, dh)
        pltpu.sync_copy(x_h.at[rsl, csl], xs.at[rsl])
        pltpu.sync_copy(z_h.at[rsl], acc.at[rsl])
        stage_wait(0, 0)
        plsc.subcore_barrier()
        for b in range(DR - 1):
            gath(0, b, b)

        def block(kk, p):
            k = 2 * kk + p
            for b in range(DR):
                bp = (b + DR - 1) % DR
                gath_wait(p, b, b)
                for g in range(CHd // 16):
                    nv16 = nrmb[p][b, pl.ds(g * 16, 16)]
                    for r in range(16):
                        i = g * 16 + r
                        sp = _dyn_gather(nv16, jnp.full((16,), r, jnp.int32))
                        for j in range(dh // 16):
                            rows[b][i, pl.ds(j * 16, 16)] = (
                                rows[b][i, pl.ds(j * 16, 16)] * sp)
                pltpu.async_copy(rows[b], acc.at[cidx[p].at[b]], ssem[b],
                                 add=True)
                if b == 0:
                    @pl.when(k > 0)
                    def _():
                        scat_wait(bp, p)

                    @pl.when(k + 1 < NB)
                    def _():
                        stage(k + 1, 1 - p)
                    gath(p, DR - 1, bp)
                else:
                    scat_wait(bp, p)
                    if b == 1:
                        @pl.when(k + 1 < NB)
                        def _():
                            stage_wait(k + 1, 1 - p)

                    @pl.when(k + 1 < NB)
                    def _():
                        gath(1 - p, b - 1, bp)

        @pl.loop(0, NB // 2)
        def dblk(kk):
            block(kk, 0)
            block(kk, 1)

        scat_wait(DR - 1, 1)
        plsc.subcore_barrier()
        pltpu.sync_copy(acc.at[rsl], out_h.at[rsl, csl])

    return prop


@functools.cache
def _make_layer(d):
    dh = d // 2
    CHd = 64 if d == 128 else CH
    T = (EPAD // NS) // CHd
    NB = T // DR
    assert NB % 2 == 0
    NR = RPT // CHd
    assert RPT % CHd == 0
    mesh = plsc.VectorSubcoreMesh(core_axis_name="c", subcore_axis_name="s")

    @functools.partial(
        pl.kernel,
        out_type=jax.ShapeDtypeStruct((4, NP, d), jnp.float32),
        mesh=mesh,
        compiler_params=pltpu.CompilerParams(use_tc_tiling_on_sc=False),
        scratch_types=[pltpu.VMEM((DR, CHd), jnp.int32) for _ in range(2)]
        + [pltpu.VMEM((DR, CHd), jnp.int32) for _ in range(2)]
        + [pltpu.VMEM((DR, CHd), jnp.float32) for _ in range(2)]
        + [pltpu.VMEM((CHd, dh), jnp.float32) for _ in range(DR)] + [
            pltpu.VMEM_SHARED((NP, dh), jnp.float32),
            pltpu.VMEM_SHARED((NP, dh), jnp.float32),
        ] + [pltpu.SemaphoreType.DMA for _ in range(2 * DR + 3)],
    )
    def layer(row_h, col_h, nrm_h, x_h, z_h, out_h,
              ri0, ri1, ci0, ci1, nr0, nr1, r0, r1, r2, r3, sa, sb,
              g0, g1, g2, g3, s0, s1, s2, s3, p0, p1, wsem):
        ridx = (ri0, ri1)
        cidx = (ci0, ci1)
        nrmb = (nr0, nr1)
        rows = (r0, r1, r2, r3)
        gsem = (g0, g1, g2, g3)
        ssem = (s0, s1, s2, s3)
        psem = (p0, p1)
        cid = lax.axis_index("c")
        sid = lax.axis_index("s")
        rsl = pl.ds(sid * RPT, RPT)
        csl = pl.ds(cid * dh, dh)

        def stage(k, p):
            pltpu.async_copy(row_h.at[sid, pl.ds(k * DR, DR)], ridx[p],
                             psem[p])
            pltpu.async_copy(col_h.at[sid, pl.ds(k * DR, DR)], cidx[p],
                             psem[p])
            pltpu.async_copy(nrm_h.at[sid, pl.ds(k * DR, DR)], nrmb[p],
                             psem[p])

        def stage_wait(k, p):
            pltpu.make_async_copy(row_h.at[sid, pl.ds(k * DR, DR)], ridx[p],
                                  psem[p]).wait()
            pltpu.make_async_copy(col_h.at[sid, pl.ds(k * DR, DR)], cidx[p],
                                  psem[p]).wait()
            pltpu.make_async_copy(nrm_h.at[sid, pl.ds(k * DR, DR)], nrmb[p],
                                  psem[p]).wait()

        def run_prop(src, acc, factor):
            def gath(p, b, rb):
                pltpu.async_copy(src.at[ridx[p].at[b]], rows[rb], gsem[rb])

            def gath_wait(p, b, rb):
                pltpu.make_async_copy(src.at[ridx[p].at[b]], rows[rb],
                                      gsem[rb]).wait()

            def scat_wait(rb, p):
                pltpu.make_async_copy(rows[rb], acc.at[cidx[p].at[0]],
                                      ssem[rb]).wait()

            stage(0, 0)
            stage_wait(0, 0)
            for b in range(DR - 1):
                gath(0, b, b)

            def block(kk, p):
                k = 2 * kk + p
                for b in range(DR):
                    bp = (b + DR - 1) % DR
                    gath_wait(p, b, b)
                    rb = rows[b]
                    nb_ = nrmb[p]

                    @pl.loop(0, CHd // 16)
                    def sgrp(g, _b=b, _rb=rb, _nb=nb_):
                        nv16 = _nb[_b, pl.ds(g * 16, 16)]
                        if factor != 1.0:
                            nv16 = nv16 * factor
                        for r in range(16):
                            i = g * 16 + r
                            sp = _dyn_gather(nv16,
                                             jnp.full((16,), r, jnp.int32))
                            for j in range(dh // 16):
                                _rb[i, pl.ds(j * 16, 16)] = (
                                    _rb[i, pl.ds(j * 16, 16)] * sp)
                    pltpu.async_copy(rows[b], acc.at[cidx[p].at[b]], ssem[b],
                                     add=True)
                    if b == 0:
                        @pl.when(k > 0)
                        def _():
                            scat_wait(bp, p)

                        @pl.when(k + 1 < NB)
                        def _():
                            stage(k + 1, 1 - p)
                        gath(p, DR - 1, bp)
                    else:
                        scat_wait(bp, p)
                        if b == 1:
                            @pl.when(k + 1 < NB)
                            def _():
                                stage_wait(k + 1, 1 - p)

                        @pl.when(k + 1 < NB)
                        def _():
                            gath(1 - p, b - 1, bp)

            @pl.loop(0, NB // 2)
            def dblk(kk):
                block(kk, 0)
                block(kk, 1)

            scat_wait(DR - 1, 1)

        pltpu.sync_copy(x_h.at[rsl, csl], sa.at[rsl])
        pltpu.sync_copy(z_h.at[rsl], sb.at[rsl])
        plsc.subcore_barrier()

        run_prop(sa, sb, 1.0)
        plsc.subcore_barrier()

        bufs = (sa, sb)
        for step in range(3):
            cur = bufs[(step + 1) % 2]
            prev = bufs[step % 2]
            wout = pltpu.async_copy(cur.at[rsl], out_h.at[step, rsl, csl],
                                    wsem)
            @pl.loop(0, NR)
            def negq(q):
                qsl = pl.ds(sid * RPT + q * CHd, CHd)
                pltpu.sync_copy(prev.at[qsl], rows[0])
                for i in range(CHd):
                    for j in range(dh // 16):
                        rows[0][i, pl.ds(j * 16, 16)] = (
                            -rows[0][i, pl.ds(j * 16, 16)])
                pltpu.sync_copy(rows[0], prev.at[qsl])
            plsc.subcore_barrier()
            run_prop(cur, prev, 2.0)
            wout.wait()
            plsc.subcore_barrier()

        pltpu.sync_copy(bufs[0].at[rsl], out_h.at[3, rsl, csl])

    return layer


@functools.cache
def _make_norm():
    T = EPW // CH
    mesh = plsc.VectorSubcoreMesh(core_axis_name="c", subcore_axis_name="s")

    NB = T // DR

    @functools.partial(
        pl.kernel,
        out_type=jax.ShapeDtypeStruct((NW, T, CH), jnp.float32),
        mesh=mesh,
        compiler_params=pltpu.CompilerParams(use_tc_tiling_on_sc=False),
        scratch_types=[
            pltpu.VMEM((T, CH), jnp.int32),
            pltpu.VMEM((T, CH), jnp.float32),
            pltpu.VMEM((T, CH), jnp.float32),
        ] + [pltpu.VMEM((CH,), jnp.float32) for _ in range(DR)]
        + [pltpu.SemaphoreType.DMA for _ in range(DR + 1)],
    )
    def normk(row_h, ew_h, dinv_h, out_h, ridx, ewv, nout,
              v0, v1, v2, v3, g0, g1, g2, g3, isem):
        dv = (v0, v1, v2, v3)
        gsem = (g0, g1, g2, g3)
        cid = lax.axis_index("c")
        sid = lax.axis_index("s")
        wid = sid * NC + cid
        d1 = pltpu.async_copy(row_h.at[wid], ridx, isem)
        d2 = pltpu.async_copy(ew_h.at[wid], ewv, isem)
        d1.wait()
        d2.wait()

        def gath(t, b):
            pltpu.async_copy(dinv_h.at[ridx.at[t]], dv[b], gsem[b])

        for b in range(DR - 1):
            gath(b, b)

        @pl.loop(0, NB)
        def blk(k):
            for b in range(DR):
                t = k * DR + b
                bp = (b + DR - 1) % DR
                pltpu.make_async_copy(dinv_h.at[ridx.at[t]], dv[b],
                                      gsem[b]).wait()
                for g in range(CH // 16):
                    sl = pl.ds(g * 16, 16)
                    nout[t, sl] = -(dv[b][sl] * ewv[t, sl])

                @pl.when(t + DR - 1 < T)
                def _():
                    gath(t + DR - 1, bp)

        pltpu.sync_copy(nout, out_h.at[wid])

    return normk


def _mm5(axs, w, b, use_sp=True, bm=512):
    di = axs[0].shape[1]
    do = w.shape[2]

    def body(a0, a1, a2, a3, a4, wr, br, o):
        aref = (a0, a1, a2, a3, a4)
        acc = jnp.dot(aref[0][...], wr[0], preferred_element_type=jnp.float32)
        for k in range(1, K):
            acc = acc + jnp.dot(aref[k][...], wr[k],
                                preferred_element_type=jnp.float32)
        acc = acc + br[...]
        if use_sp:
            acc = _softplus(acc)
        o[...] = acc

    return pl.pallas_call(
        body,
        grid=(NP // bm,),
        in_specs=[pl.BlockSpec((bm, di), lambda i: (i, 0)) for _ in range(K)]
        + [pl.BlockSpec((K, di, do), lambda i: (0, 0, 0)),
           pl.BlockSpec((1, do), lambda i: (0, 0))],
        out_specs=pl.BlockSpec((bm, do), lambda i: (i, 0)),
        out_shape=jax.ShapeDtypeStruct((NP, do), jnp.float32),
    )(*axs, w, b.reshape(1, do))


def _mm1(a, w, b=None, use_sp=False, bm=512):
    di = a.shape[1]
    do = w.shape[1]

    def body(ar, wr, br, o):
        acc = jnp.dot(ar[...], wr[...], preferred_element_type=jnp.float32)
        acc = acc + br[...]
        if use_sp:
            acc = _softplus(acc)
        o[...] = acc

    if b is None:
        b = jnp.zeros((do,), jnp.float32)
    return pl.pallas_call(
        body,
        grid=(NP // bm,),
        in_specs=[pl.BlockSpec((bm, di), lambda i: (i, 0)),
                  pl.BlockSpec((di, do), lambda i: (0, 0)),
                  pl.BlockSpec((1, do), lambda i: (0, 0))],
        out_specs=pl.BlockSpec((bm, do), lambda i: (i, 0)),
        out_shape=jax.ShapeDtypeStruct((NP, do), jnp.float32),
    )(a, w, b.reshape(1, do))


def _comb(coefs, xs, b=None, use_sp=False, bm=1024):
    nx = len(coefs)
    d = xs[0].shape[1]
    use_bias = b is not None

    def body(*refs):
        o = refs[-1]
        acc = coefs[0] * refs[0][...]
        for j in range(1, nx):
            acc = acc + coefs[j] * refs[j][...]
        if use_bias:
            acc = acc + refs[nx][...]
        if use_sp:
            acc = _softplus(acc)
        o[...] = acc

    in_specs = [pl.BlockSpec((bm, d), lambda i: (i, 0)) for _ in range(nx)]
    args = list(xs)
    if use_bias:
        in_specs.append(pl.BlockSpec((1, d), lambda i: (0, 0)))
        args.append(b.reshape(1, d))
    return pl.pallas_call(
        body,
        grid=(NP // bm,),
        in_specs=in_specs,
        out_specs=pl.BlockSpec((bm, d), lambda i: (i, 0)),
        out_shape=jax.ShapeDtypeStruct((NP, d), jnp.float32),
    )(*args)


def _e3(a, chd=CH):
    return a.reshape(NW, EPW // chd // 2, chd) if chd == CH else a


def _es(a, chd):
    return a.reshape(NS, (EPAD // NS) // chd, chd)


def _propagate(rowp, colp, nrm, h):
    d = h.shape[1]
    if d == 16:
        args = [a.reshape(NW, EPW // CH, CH) for a in (rowp, colp, nrm)]
        part = _make_prop(16)(*args, h, jnp.zeros((NP, 16), jnp.float32))
        return _comb((1.0, 1.0), (part[0], part[1]))
    outs = []
    dep = None
    for lo in range(0, d, 128):
        w = min(128, d - lo)
        hin = h[:, lo:lo + w]
        if dep is not None:
            hin, _ = lax.optimization_barrier((hin, dep))
        chd = 64 if w == 128 else CH
        args = [_es(a, chd) for a in (rowp, colp, nrm)]
        s = _make_prop_fs(w)(*args, hin,
                             jnp.zeros((NP, w // 2), jnp.float32))
        dep = s
        outs.append(s)
    return outs[0] if len(outs) == 1 else jnp.concatenate(outs, axis=1)


def kernel(x, edge_weigth, params, edge_index, batch):
    row = edge_index[0]
    col = edge_index[1]
    pad = EPAD - E
    padidx = (jnp.arange(pad, dtype=jnp.int32) * 37) % N
    rowp = jnp.concatenate([row, padidx])
    colp = jnp.concatenate([col, padidx])
    ewp = jnp.concatenate([edge_weigth, jnp.zeros((pad,), jnp.float32)])
    r3 = rowp.reshape(NW, EPW // CH, CH)
    c3 = colp.reshape(NW, EPW // CH, CH)
    w3 = ewp.reshape(NW, EPW // CH, CH)

    x = jnp.concatenate([x, jnp.zeros((NP - N, x.shape[1]), jnp.float32)])
    ones16 = jnp.ones((NP, 16), jnp.float32)
    z16 = jnp.zeros((NP, 16), jnp.float32)
    dparts = _make_prop(16)(r3, r3, w3, ones16, z16)
    deg = dparts[0, :, 0] + dparts[1, :, 0]
    dinv = jnp.where(deg > 0, 1.0 / deg, 0.0)
    nrm = _make_norm()(r3, w3, dinv)
    nrmf = nrm.reshape(-1)
    n3 = nrm

    w0 = params["W0"]
    u = _mm1(x, w0.transpose(1, 0, 2).reshape(128, K * 16))
    u = [u[:, 16 * k:16 * (k + 1)] for k in range(K)]
    b4 = u[4]
    p = _make_prop(16)(r3, c3, n3, b4, z16)
    b3 = _comb((2.0, 2.0, 1.0), (p[0], p[1], u[3]))
    p = _make_prop(16)(r3, c3, n3, b3, z16)
    b2 = _comb((2.0, 2.0, -1.0, 1.0), (p[0], p[1], b4, u[2]))
    p = _make_prop(16)(r3, c3, n3, b2, z16)
    b1 = _comb((2.0, 2.0, -1.0, 1.0), (p[0], p[1], b3, u[1]))
    p = _make_prop(16)(r3, c3, n3, b1, z16)
    h = _comb((1.0, 1.0, -1.0, 1.0), (p[0], p[1], b2, u[0]),
              b=params["b0"], use_sp=True)

    for i in range(1, 7):
        wi = params["W%d" % i]
        tx0 = h
        di = tx0.shape[1]
        if di == 16 or di >= 128:
            tx1 = _propagate(rowp, colp, nrmf, tx0)
            s = _propagate(rowp, colp, nrmf, tx1)
            tx2 = _comb((2.0, -1.0), (s, tx0))
            s = _propagate(rowp, colp, nrmf, tx2)
            tx3 = _comb((2.0, -1.0), (s, tx1))
            s = _propagate(rowp, colp, nrmf, tx3)
            tx4 = _comb((2.0, -1.0), (s, tx2))
            txs = (tx1, tx2, tx3, tx4)
        else:
            groups = []
            dep = None
            for lo in range(0, di, 128):
                w = min(128, di - lo)
                hin = tx0[:, lo:lo + w]
                if dep is not None:
                    hin, _ = lax.optimization_barrier((hin, dep))
                chd = 64 if w == 128 else CH
                args = [_es(a, chd) for a in (rowp, colp, nrmf)]
                g = _make_layer(w)(*args, hin,
                                   jnp.zeros((NP, w // 2), jnp.float32))
                dep = g
                groups.append(g)
            if len(groups) == 1:
                txs = tuple(groups[0][k] for k in range(4))
            else:
                txs = tuple(
                    jnp.concatenate([g[k] for g in groups], axis=1)
                    for k in range(4))
        h = _mm5((tx0,) + txs, wi, params["b%d" % i], use_sp=True)

    fcw = jnp.zeros((512, 128), jnp.float32).at[:, :3].set(params["fc_w"].T)
    fcb = jnp.zeros((128,), jnp.float32).at[:3].set(params["fc_b"])
    out = _mm1(h, fcw, fcb)
    return out[:N, :3]
